# baseline (speedup 1.0000x reference)
import functools

import jax
import jax.numpy as jnp
from jax import lax
from jax.experimental import pallas as pl
from jax.experimental.pallas import tpu as pltpu
from jax.experimental.pallas import tpu_sc as plsc


@functools.lru_cache(maxsize=None)
def _make_sc_part(B, N, D, B1, bpw, bpt, lanes):
    mesh = plsc.VectorSubcoreMesh(core_axis_name="c", subcore_axis_name="s")
    n_cores = 2
    nd = D // lanes

    @functools.partial(
        pl.kernel,
        mesh=mesh,
        out_type=[
            jax.ShapeDtypeStruct((B, D), jnp.float32),
            jax.ShapeDtypeStruct((B - B1, D), jnp.float32),
        ],
        scratch_types=[
            pltpu.VMEM((bpw,), jnp.int32),
            pltpu.VMEM((bpw, D), jnp.float32),
            pltpu.VMEM((N, D), jnp.float32),
            pltpu.VMEM((bpt, D), jnp.float32),
            pltpu.SemaphoreType.DMA,
        ],
    )
    def sc_part(table_hbm, idx_hbm, prev_hbm, mean_hbm,
                idx_v, rows_v, ebuf, obuf, sem):
        wid = lax.axis_index("s") * n_cores + lax.axis_index("c")
        base = wid * bpw
        pltpu.sync_copy(idx_hbm.at[pl.ds(base, bpw)], idx_v)
        lane = lax.iota(jnp.int32, lanes)
        for j in range(bpw // lanes):
            i_vec = base + j * lanes + lane
            idx_v[pl.ds(j * lanes, lanes)] = (
                i_vec * N + idx_v[pl.ds(j * lanes, lanes)]
            )
        pltpu.async_copy(table_hbm.at[idx_v], rows_v, sem).wait()
        pltpu.sync_copy(rows_v, prev_hbm.at[pl.ds(base, bpw)])

        mb = B1 + wid * bpt
        inv_n = jnp.float32(1.0 / N)

        def batch_body(i, carry):
            pltpu.sync_copy(table_hbm.at[pl.ds((mb + i) * N, N)], ebuf)

            def row_body(j, acc):
                return tuple(
                    acc[k] + ebuf[j, pl.ds(k * lanes, lanes)]
                    for k in range(nd)
                )

            acc0 = tuple(jnp.zeros((lanes,), jnp.float32) for _ in range(nd))
            acc = lax.fori_loop(0, N, row_body, acc0)
            for k in range(nd):
                obuf[i, pl.ds(k * lanes, lanes)] = acc[k] * inv_n
            return carry

        lax.fori_loop(0, bpt, batch_body, 0)
        pltpu.sync_copy(obuf, mean_hbm.at[pl.ds(wid * bpt, bpt)])

    return sc_part


def _mean_body(inv_n, emb_ref, out_ref):
    out_ref[...] = jnp.sum(emb_ref[...], axis=1) * inv_n


@functools.lru_cache(maxsize=None)
def _make_tc_mean(B1, N, D, bb):
    return pl.pallas_call(
        functools.partial(_mean_body, 1.0 / N),
        grid=(B1 // bb,),
        in_specs=[pl.BlockSpec((bb, N, D), lambda i: (i, 0, 0))],
        out_specs=pl.BlockSpec((bb, D), lambda i: (i, 0)),
        out_shape=jax.ShapeDtypeStruct((B1, D), jnp.float32),
        compiler_params=pltpu.CompilerParams(
            dimension_semantics=("arbitrary",),
        ),
    )


def _combine_body(nb1, prev_ref, mtc_ref, msc_ref, out_ref):
    d = prev_ref.shape[-1]
    i = pl.program_id(0)
    out_ref[:, :d] = prev_ref[...]
    out_ref[:, d:] = jnp.where(i < nb1, mtc_ref[...], msc_ref[...])


@functools.lru_cache(maxsize=None)
def _make_tc_combine(B, B1, D, bb):
    nb1 = B1 // bb
    return pl.pallas_call(
        functools.partial(_combine_body, nb1),
        grid=(B // bb,),
        in_specs=[
            pl.BlockSpec((bb, D), lambda i: (i, 0)),
            pl.BlockSpec((bb, D), lambda i: (jnp.minimum(i, nb1 - 1), 0)),
            pl.BlockSpec((bb, D), lambda i: (jnp.maximum(i - nb1, 0), 0)),
        ],
        out_specs=pl.BlockSpec((bb, 2 * D), lambda i: (i, 0)),
        out_shape=jax.ShapeDtypeStruct((B, 2 * D), jnp.float32),
        compiler_params=pltpu.CompilerParams(
            dimension_semantics=("arbitrary",),
        ),
    )


def kernel(embeddings, current_node):
    B, N, D = embeddings.shape
    nw, lanes = 32, 16
    B2 = 512
    B1 = B - B2
    bpw = B // nw
    bpt = B2 // nw

    table = embeddings.reshape(B * N, D)
    idx = current_node.reshape(B).astype(jnp.int32)

    prev, mean_sc = _make_sc_part(B, N, D, B1, bpw, bpt, lanes)(table, idx)
    mean_tc = _make_tc_mean(B1, N, D, 64)(embeddings)
    out = _make_tc_combine(B, B1, D, 512)(prev, mean_tc, mean_sc)
    return out.reshape(B, 1, 2 * D)
